# Initial kernel scaffold; baseline (speedup 1.0000x reference)
import jax
import jax.numpy as jnp
from jax.experimental import pallas as pl


def kernel(key_embed, key_phi):
    raise NotImplementedError("write your pallas kernel here")


# speedup vs baseline: 1.3171x; 1.3171x over previous
import functools

import jax
import jax.numpy as jnp
from jax import lax
from jax.experimental import pallas as pl
from jax.experimental.pallas import tpu as pltpu
from jax.experimental.pallas import tpu_sc as plsc

B, N, D = 8, 4096, 256
BLK = 512
NW = 32
ROWS_PER_W = (B * N) // NW
CHUNK = 128
NCHUNK = ROWS_PER_W // CHUNK


def _rank_body(phi_row_ref, phi_col_ref, fidx_ref):
    blk = pl.program_id(0)
    phi_i = phi_row_ref[...]
    phi_j = phi_col_ref[...]
    jidx = lax.broadcasted_iota(jnp.int32, (N, BLK), 0)
    iidx = lax.broadcasted_iota(jnp.int32, (N, BLK), 1) + blk * BLK
    less = (phi_j < phi_i) | ((phi_j == phi_i) & (jidx < iidx))
    rank = jnp.sum(less.astype(jnp.int32), axis=0, keepdims=True)
    boff = lax.broadcasted_iota(jnp.int32, (B, BLK), 0) * N
    fidx_ref[...] = jnp.broadcast_to(rank, (B, BLK)) + boff


def _compute_flat_idx(phi_row, phi_col):
    return pl.pallas_call(
        _rank_body,
        grid=(N // BLK,),
        in_specs=[
            pl.BlockSpec((1, BLK), lambda b: (0, b)),
            pl.BlockSpec((N, 1), lambda b: (0, 0)),
        ],
        out_specs=pl.BlockSpec((B, BLK), lambda b: (0, b)),
        out_shape=jax.ShapeDtypeStruct((B, N), jnp.int32),
    )(phi_row, phi_col)


@functools.cache
def _make_sc_scatter():
    mesh = plsc.VectorSubcoreMesh(core_axis_name="c", subcore_axis_name="s")

    @functools.partial(
        pl.kernel,
        mesh=mesh,
        out_type=[
            jax.ShapeDtypeStruct((B * N, D), jnp.float32),
            jax.ShapeDtypeStruct((B * N,), jnp.float32),
        ],
        scratch_types=[
            pltpu.VMEM((NCHUNK, CHUNK), jnp.int32),
            pltpu.VMEM((NCHUNK, CHUNK), jnp.float32),
            pltpu.VMEM((CHUNK, D), jnp.float32),
            pltpu.VMEM((CHUNK, D), jnp.float32),
            pltpu.SemaphoreType.DMA,
            pltpu.SemaphoreType.DMA,
        ],
    )
    def _sc_scatter(embed_hbm, fidx_hbm, phi_hbm, out_embed, out_phi,
                    idx_v, phi_v, buf0, buf1, sem_emb, sem_phi):
        wid = lax.axis_index("s") * 2 + lax.axis_index("c")
        rowbase = wid * ROWS_PER_W
        pltpu.sync_copy(fidx_hbm.at[pl.ds(wid * NCHUNK, NCHUNK)], idx_v)
        pltpu.sync_copy(phi_hbm.at[pl.ds(wid * NCHUNK, NCHUNK)], phi_v)
        for c in range(NCHUNK):
            buf = buf0 if c % 2 == 0 else buf1
            pltpu.sync_copy(embed_hbm.at[pl.ds(rowbase + c * CHUNK, CHUNK)],
                            buf)
            pltpu.async_copy(buf, out_embed.at[idx_v.at[c]], sem_emb).wait()
            pltpu.async_copy(phi_v.at[c], out_phi.at[idx_v.at[c]],
                             sem_phi).wait()

    return _sc_scatter


def kernel(key_embed, key_phi):
    phi_row = key_phi[0:1, :]
    phi_col = key_phi[0].reshape(N, 1)
    fidx = _compute_flat_idx(phi_row, phi_col)
    embed_flat = key_embed.reshape(B * N, D)
    fidx2d = fidx.reshape((B * N) // CHUNK, CHUNK)
    phi2d = key_phi.reshape((B * N) // CHUNK, CHUNK)
    emb_sorted, phi_sorted = _make_sc_scatter()(embed_flat, fidx2d, phi2d)
    return emb_sorted.reshape(B, N, D), phi_sorted.reshape(B, N)


# speedup vs baseline: 1.3546x; 1.0284x over previous
import functools

import jax
import jax.numpy as jnp
from jax import lax
from jax.experimental import pallas as pl
from jax.experimental.pallas import tpu as pltpu
from jax.experimental.pallas import tpu_sc as plsc

B, N, D = 8, 4096, 256
BLK = 512
NW = 32
ROWS_PER_W = (B * N) // NW
CHUNK = 128
NCHUNK = ROWS_PER_W // CHUNK


def _rank_body(phi_row_ref, phi_col_ref, fidx_ref):
    blk = pl.program_id(0)
    phi_i = phi_row_ref[...]
    phi_j = phi_col_ref[...]
    jidx = lax.broadcasted_iota(jnp.int32, (N, BLK), 0)
    iidx = lax.broadcasted_iota(jnp.int32, (N, BLK), 1) + blk * BLK
    less = (phi_j < phi_i) | ((phi_j == phi_i) & (jidx < iidx))
    rank = jnp.sum(less.astype(jnp.int32), axis=0, keepdims=True)
    boff = lax.broadcasted_iota(jnp.int32, (B, BLK), 0) * N
    fidx_ref[...] = jnp.broadcast_to(rank, (B, BLK)) + boff


def _compute_flat_idx(phi_row, phi_col):
    return pl.pallas_call(
        _rank_body,
        grid=(N // BLK,),
        in_specs=[
            pl.BlockSpec((1, BLK), lambda b: (0, b)),
            pl.BlockSpec((N, 1), lambda b: (0, 0)),
        ],
        out_specs=pl.BlockSpec((B, BLK), lambda b: (0, b)),
        out_shape=jax.ShapeDtypeStruct((B, N), jnp.int32),
    )(phi_row, phi_col)


@functools.cache
def _make_sc_scatter():
    mesh = plsc.VectorSubcoreMesh(core_axis_name="c", subcore_axis_name="s")

    @functools.partial(
        pl.kernel,
        mesh=mesh,
        out_type=[
            jax.ShapeDtypeStruct((B * N, D), jnp.float32),
            jax.ShapeDtypeStruct((B * N,), jnp.float32),
        ],
        scratch_types=[
            pltpu.VMEM((NCHUNK, CHUNK), jnp.int32),
            pltpu.VMEM((NCHUNK, CHUNK), jnp.float32),
            pltpu.VMEM((CHUNK, D), jnp.float32),
            pltpu.VMEM((CHUNK, D), jnp.float32),
            pltpu.VMEM((CHUNK, D), jnp.float32),
            pltpu.SemaphoreType.DMA,
            pltpu.SemaphoreType.DMA,
            pltpu.SemaphoreType.DMA,
            pltpu.SemaphoreType.DMA,
            pltpu.SemaphoreType.DMA,
            pltpu.SemaphoreType.DMA,
            pltpu.SemaphoreType.DMA,
        ],
    )
    def _sc_scatter(embed_hbm, fidx_hbm, phi_hbm, out_embed, out_phi,
                    idx_v, phi_v, buf0, buf1, buf2,
                    sl0, sl1, sl2, ss0, ss1, ss2, sem_phi):
        bufs = (buf0, buf1, buf2)
        sem_ld = (sl0, sl1, sl2)
        sem_st = (ss0, ss1, ss2)
        wid = lax.axis_index("s") * 2 + lax.axis_index("c")
        rowbase = wid * ROWS_PER_W
        pltpu.sync_copy(fidx_hbm.at[pl.ds(wid * NCHUNK, NCHUNK)], idx_v)
        pltpu.sync_copy(phi_hbm.at[pl.ds(wid * NCHUNK, NCHUNK)], phi_v)

        def load(c):
            return pltpu.async_copy(
                embed_hbm.at[pl.ds(rowbase + c * CHUNK, CHUNK)],
                bufs[c % 3], sem_ld[c % 3])

        loads = [None] * NCHUNK
        scats = [None] * NCHUNK
        phis = [None] * NCHUNK
        loads[0] = load(0)
        loads[1] = load(1)
        for c in range(NCHUNK):
            loads[c].wait()
            scats[c] = pltpu.async_copy(
                bufs[c % 3], out_embed.at[idx_v.at[c]], sem_st[c % 3])
            phis[c] = pltpu.async_copy(
                phi_v.at[c], out_phi.at[idx_v.at[c]], sem_phi)
            if c + 2 < NCHUNK:
                if c >= 1:
                    scats[c - 1].wait()
                loads[c + 2] = load(c + 2)
        for c in range(NCHUNK - 3, NCHUNK):
            scats[c].wait()
        for c in range(NCHUNK):
            phis[c].wait()

    return _sc_scatter


def kernel(key_embed, key_phi):
    phi_row = key_phi[0:1, :]
    phi_col = key_phi[0].reshape(N, 1)
    fidx = _compute_flat_idx(phi_row, phi_col)
    embed_flat = key_embed.reshape(B * N, D)
    fidx2d = fidx.reshape((B * N) // CHUNK, CHUNK)
    phi2d = key_phi.reshape((B * N) // CHUNK, CHUNK)
    emb_sorted, phi_sorted = _make_sc_scatter()(embed_flat, fidx2d, phi2d)
    return emb_sorted.reshape(B, N, D), phi_sorted.reshape(B, N)


# speedup vs baseline: 1.4359x; 1.0600x over previous
import functools

import jax
import jax.numpy as jnp
from jax import lax
from jax.experimental import pallas as pl
from jax.experimental.pallas import tpu as pltpu
from jax.experimental.pallas import tpu_sc as plsc

B, N, D = 8, 4096, 256
BLK = 512
NW = 32
ROWS_PER_W = (B * N) // NW
CHUNK = 128
NCHUNK = ROWS_PER_W // CHUNK


def _rank_body(phi_ref, phi_col_ref, fidx_ref, phis_ref):
    blk = pl.program_id(0)
    phi_blk = phi_ref[...]
    phi_i = phi_blk[0:1, :]
    phi_j = phi_col_ref[...]
    jidx = lax.broadcasted_iota(jnp.int32, (N, BLK), 0)
    iidx = lax.broadcasted_iota(jnp.int32, (N, BLK), 1) + blk * BLK
    less = (phi_j < phi_i) | ((phi_j == phi_i) & (jidx < iidx))
    rank = jnp.sum(less.astype(jnp.int32), axis=0, keepdims=True)
    boff = lax.broadcasted_iota(jnp.int32, (B, BLK), 0) * N
    fidx_ref[...] = jnp.broadcast_to(rank, (B, BLK)) + boff
    onehot_t = (jidx == rank).astype(jnp.float32)
    contrib = lax.dot_general(
        phi_blk, onehot_t, (((1,), (1,)), ((), ())),
        precision=lax.Precision.HIGHEST,
        preferred_element_type=jnp.float32)

    @pl.when(blk == 0)
    def _():
        phis_ref[...] = contrib

    @pl.when(blk > 0)
    def _():
        phis_ref[...] += contrib


def _rank_and_phi(key_phi, phi_col):
    return pl.pallas_call(
        _rank_body,
        grid=(N // BLK,),
        in_specs=[
            pl.BlockSpec((B, BLK), lambda b: (0, b)),
            pl.BlockSpec((N, 1), lambda b: (0, 0)),
        ],
        out_specs=[
            pl.BlockSpec((B, BLK), lambda b: (0, b)),
            pl.BlockSpec((B, N), lambda b: (0, 0)),
        ],
        out_shape=[
            jax.ShapeDtypeStruct((B, N), jnp.int32),
            jax.ShapeDtypeStruct((B, N), jnp.float32),
        ],
    )(key_phi, phi_col)


@functools.cache
def _make_sc_scatter():
    mesh = plsc.VectorSubcoreMesh(core_axis_name="c", subcore_axis_name="s")

    @functools.partial(
        pl.kernel,
        mesh=mesh,
        out_type=jax.ShapeDtypeStruct((B * N, D), jnp.float32),
        scratch_types=[
            pltpu.VMEM((NCHUNK, CHUNK), jnp.int32),
            pltpu.VMEM((CHUNK, D), jnp.float32),
            pltpu.VMEM((CHUNK, D), jnp.float32),
            pltpu.VMEM((CHUNK, D), jnp.float32),
            pltpu.SemaphoreType.DMA,
            pltpu.SemaphoreType.DMA,
            pltpu.SemaphoreType.DMA,
            pltpu.SemaphoreType.DMA,
            pltpu.SemaphoreType.DMA,
            pltpu.SemaphoreType.DMA,
        ],
    )
    def _sc_scatter(embed_hbm, fidx_hbm, out_embed,
                    idx_v, buf0, buf1, buf2,
                    sl0, sl1, sl2, ss0, ss1, ss2):
        bufs = (buf0, buf1, buf2)
        sem_ld = (sl0, sl1, sl2)
        sem_st = (ss0, ss1, ss2)
        wid = lax.axis_index("s") * 2 + lax.axis_index("c")
        rowbase = wid * ROWS_PER_W
        pltpu.sync_copy(fidx_hbm.at[pl.ds(wid * NCHUNK, NCHUNK)], idx_v)

        def load(c):
            return pltpu.async_copy(
                embed_hbm.at[pl.ds(rowbase + c * CHUNK, CHUNK)],
                bufs[c % 3], sem_ld[c % 3])

        loads = [None] * NCHUNK
        scats = [None] * NCHUNK
        loads[0] = load(0)
        loads[1] = load(1)
        for c in range(NCHUNK):
            loads[c].wait()
            scats[c] = pltpu.async_copy(
                bufs[c % 3], out_embed.at[idx_v.at[c]], sem_st[c % 3])
            if c + 2 < NCHUNK:
                if c >= 1:
                    scats[c - 1].wait()
                loads[c + 2] = load(c + 2)
        for c in range(NCHUNK - 3, NCHUNK):
            scats[c].wait()

    return _sc_scatter


def kernel(key_embed, key_phi):
    phi_col = key_phi[0].reshape(N, 1)
    fidx, phi_sorted = _rank_and_phi(key_phi, phi_col)
    embed_flat = key_embed.reshape(B * N, D)
    fidx2d = fidx.reshape((B * N) // CHUNK, CHUNK)
    emb_sorted = _make_sc_scatter()(embed_flat, fidx2d)
    return emb_sorted.reshape(B, N, D), phi_sorted


# speedup vs baseline: 2.2884x; 1.5937x over previous
import functools

import jax
import jax.numpy as jnp
from jax import lax
from jax.experimental import pallas as pl
from jax.experimental.pallas import tpu as pltpu
from jax.experimental.pallas import tpu_sc as plsc

B, N, D = 8, 4096, 256
BLK = 512
NW = 32
ROWS_PER_W = (B * N) // NW
CHUNK = 128
NCHUNK = ROWS_PER_W // CHUNK
PHI_PER_W = N // NW


def _rank_body(phi_row_ref, phi_col_ref, fidx_ref):
    blk = pl.program_id(0)
    phi_i = phi_row_ref[...]
    phi_j = phi_col_ref[...]
    jidx = lax.broadcasted_iota(jnp.int32, (N, BLK), 0)
    iidx = lax.broadcasted_iota(jnp.int32, (N, BLK), 1) + blk * BLK
    less = (phi_j < phi_i) | ((phi_j == phi_i) & (jidx < iidx))
    rank = jnp.sum(less.astype(jnp.int32), axis=0, keepdims=True)
    boff = lax.broadcasted_iota(jnp.int32, (B, BLK), 0) * N
    fidx_ref[...] = jnp.broadcast_to(rank, (B, BLK)) + boff


def _compute_flat_idx(phi_row, phi_col):
    return pl.pallas_call(
        _rank_body,
        grid=(N // BLK,),
        in_specs=[
            pl.BlockSpec((1, BLK), lambda b: (0, b)),
            pl.BlockSpec((N, 1), lambda b: (0, 0)),
        ],
        out_specs=pl.BlockSpec((B, BLK), lambda b: (0, b)),
        out_shape=jax.ShapeDtypeStruct((B, N), jnp.int32),
    )(phi_row, phi_col)


@functools.cache
def _make_sc_scatter():
    mesh = plsc.VectorSubcoreMesh(core_axis_name="c", subcore_axis_name="s")

    @functools.partial(
        pl.kernel,
        mesh=mesh,
        out_type=[
            jax.ShapeDtypeStruct((B * N, D), jnp.float32),
            jax.ShapeDtypeStruct((N, 128), jnp.float32),
        ],
        scratch_types=[
            pltpu.VMEM((NCHUNK, CHUNK), jnp.int32),
            pltpu.VMEM((1, PHI_PER_W), jnp.int32),
            pltpu.VMEM((PHI_PER_W, 128), jnp.float32),
            pltpu.VMEM((CHUNK, D), jnp.float32),
            pltpu.VMEM((CHUNK, D), jnp.float32),
            pltpu.VMEM((CHUNK, D), jnp.float32),
            pltpu.SemaphoreType.DMA,
            pltpu.SemaphoreType.DMA,
            pltpu.SemaphoreType.DMA,
            pltpu.SemaphoreType.DMA,
            pltpu.SemaphoreType.DMA,
            pltpu.SemaphoreType.DMA,
            pltpu.SemaphoreType.DMA,
        ],
    )
    def _sc_scatter(embed_hbm, fidx_hbm, phit_hbm, out_embed, out_phit,
                    idx_v, idxp_v, phi_buf, buf0, buf1, buf2,
                    sl0, sl1, sl2, ss0, ss1, ss2, sem_phi):
        bufs = (buf0, buf1, buf2)
        sem_ld = (sl0, sl1, sl2)
        sem_st = (ss0, ss1, ss2)
        wid = lax.axis_index("s") * 2 + lax.axis_index("c")
        rowbase = wid * ROWS_PER_W
        pltpu.sync_copy(fidx_hbm.at[pl.ds(wid * NCHUNK, NCHUNK)], idx_v)
        pltpu.sync_copy(fidx_hbm.at[pl.ds(wid, 1)], idxp_v)
        pltpu.sync_copy(phit_hbm.at[pl.ds(wid * PHI_PER_W, PHI_PER_W)],
                        phi_buf)
        phi_scat = pltpu.async_copy(
            phi_buf, out_phit.at[idxp_v.at[0]], sem_phi)

        def load(c):
            return pltpu.async_copy(
                embed_hbm.at[pl.ds(rowbase + c * CHUNK, CHUNK)],
                bufs[c % 3], sem_ld[c % 3])

        loads = [None] * NCHUNK
        scats = [None] * NCHUNK
        loads[0] = load(0)
        loads[1] = load(1)
        for c in range(NCHUNK):
            loads[c].wait()
            scats[c] = pltpu.async_copy(
                bufs[c % 3], out_embed.at[idx_v.at[c]], sem_st[c % 3])
            if c + 2 < NCHUNK:
                if c >= 1:
                    scats[c - 1].wait()
                loads[c + 2] = load(c + 2)
        for c in range(NCHUNK - 3, NCHUNK):
            scats[c].wait()
        phi_scat.wait()

    return _sc_scatter


def kernel(key_embed, key_phi):
    phi_row = key_phi[0:1, :]
    phi_col = key_phi[0].reshape(N, 1)
    fidx = _compute_flat_idx(phi_row, phi_col)
    embed_flat = key_embed.reshape(B * N, D)
    fidx2d = fidx.reshape((B * N) // CHUNK, CHUNK)
    phit = jnp.pad(key_phi.T, ((0, 0), (0, 128 - B)))
    emb_sorted, phit_sorted = _make_sc_scatter()(embed_flat, fidx2d, phit)
    return emb_sorted.reshape(B, N, D), phit_sorted[:, :B].T
